# baseline (speedup 1.0000x reference)
import functools

import jax
import jax.numpy as jnp
from jax import lax
from jax.experimental import pallas as pl
from jax.experimental.pallas import tpu as pltpu
from jax.experimental.pallas import tpu_sc as plsc

_N, _F, _H, _C = 10000, 128, 128, 10
_E = 320000
_NC, _NS = 2, 16
_NW = _NC * _NS
_EPT = _E // _NW
_CHUNK = 64
_NCH = _EPT // _CHUNK
_TAIL = _EPT - _NCH * _CHUNK
_NPAD = 10112
_RPT = _NPAD // _NS
_NRING = 5
_NQ = _NCH // _NRING


def _agg_body(feat, src, dst, out,
              si0, si1, si2, si3, si4, sit, di0, di1, di2, di3, di4, dit,
              rows0, rows1, rows2, rows3, rows4, rowst,
              acc, gsem, ssem, isem):
    c = lax.axis_index("c")
    s = lax.axis_index("s")
    base = (c * _NS + s) * _EPT
    sis = (si0, si1, si2, si3, si4)
    dis = (di0, di1, di2, di3, di4)
    rows = (rows0, rows1, rows2, rows3, rows4)

    def sslice(k):
        return src.at[pl.ds(base + k * _CHUNK, _CHUNK)]

    def dslice(k):
        return dst.at[pl.ds(base + k * _CHUNK, _CHUNK)]

    for b in range(_NRING):
        pltpu.async_copy(sslice(b), sis[b], isem)
    for b in range(_NRING):
        pltpu.async_copy(dslice(b), dis[b], isem)
    for b in range(_NRING - 1):
        pltpu.make_async_copy(sslice(b), sis[b], isem).wait()
        pltpu.async_copy(feat.at[sis[b]], rows[b], gsem)

    zv = jnp.zeros((16,), jnp.float32)
    zbuf = rows[_NRING - 1]

    def zrow(r, carry):
        for k in range(8):
            zbuf[r, pl.ds(k * 16, 16)] = zv
        return carry

    lax.fori_loop(0, _CHUNK, zrow, 0)
    for t in range(_RPT // _CHUNK):
        pltpu.async_copy(zbuf, acc.at[pl.ds(s * _RPT + t * _CHUNK, _CHUNK)],
                         ssem)
    _nz = _RPT - (_RPT // _CHUNK) * _CHUNK
    pltpu.async_copy(zbuf.at[pl.ds(0, _nz)],
                     acc.at[pl.ds(s * _RPT + _RPT - _nz, _nz)], ssem)
    for t in range(_RPT // _CHUNK):
        pltpu.make_async_copy(
            zbuf, acc.at[pl.ds(s * _RPT + t * _CHUNK, _CHUNK)], ssem).wait()
    pltpu.make_async_copy(
        zbuf.at[pl.ds(0, _nz)],
        acc.at[pl.ds(s * _RPT + _RPT - _nz, _nz)], ssem).wait()
    plsc.subcore_barrier()

    pltpu.make_async_copy(sslice(_NRING - 1), sis[_NRING - 1], isem).wait()
    pltpu.async_copy(feat.at[sis[_NRING - 1]], rows[_NRING - 1], gsem)

    def round_(i, carry):
        j0 = _NRING * i
        for b in range(_NRING):
            k = j0 + b
            pltpu.make_async_copy(dslice(k), dis[b], isem).wait()
            pltpu.make_async_copy(feat.at[sis[b]], rows[b], gsem).wait()
            pltpu.async_copy(rows[b], acc.at[dis[b]], ssem, add=True)

            @pl.when(k + _NRING < _NCH)
            def _load_src(b=b, k=k):
                pltpu.async_copy(sslice(k + _NRING), sis[b], isem)

        for b in range(_NRING):
            k = j0 + b
            pltpu.make_async_copy(rows[b], acc.at[dis[b]], ssem).wait()

            @pl.when(k + _NRING < _NCH)
            def _issue(b=b, k=k):
                pltpu.async_copy(dslice(k + _NRING), dis[b], isem)
                pltpu.make_async_copy(sslice(k + _NRING), sis[b],
                                      isem).wait()
                pltpu.async_copy(feat.at[sis[b]], rows[b], gsem)

        return carry

    lax.fori_loop(0, _NQ, round_, 0)

    kl = _NQ * _NRING
    bl = kl % _NRING
    pltpu.make_async_copy(dslice(kl), dis[bl], isem).wait()
    pltpu.make_async_copy(feat.at[sis[bl]], rows[bl], gsem).wait()
    pltpu.async_copy(rows[bl], acc.at[dis[bl]], ssem, add=True)

    ot = base + _NCH * _CHUNK
    pltpu.sync_copy(src.at[pl.ds(ot, _TAIL)], sit)
    pltpu.sync_copy(dst.at[pl.ds(ot, _TAIL)], dit)
    pltpu.async_copy(feat.at[sit], rowst, gsem).wait()
    pltpu.sync_copy(rowst, acc.at[dit], add=True)
    pltpu.make_async_copy(rows[bl], acc.at[dis[bl]], ssem).wait()

    plsc.subcore_barrier()
    pltpu.sync_copy(acc.at[pl.ds(s * _RPT, _RPT)],
                    out.at[c, pl.ds(s * _RPT, _RPT)])


@functools.cache
def _make_sc_agg():
    return pl.kernel(
        _agg_body,
        out_type=jax.ShapeDtypeStruct((_NC, _NPAD, _F), jnp.float32),
        mesh=plsc.VectorSubcoreMesh(core_axis_name="c", subcore_axis_name="s",
                                    num_cores=_NC, num_subcores=_NS),
        scratch_types=[
            pltpu.VMEM((_CHUNK,), jnp.int32),
            pltpu.VMEM((_CHUNK,), jnp.int32),
            pltpu.VMEM((_CHUNK,), jnp.int32),
            pltpu.VMEM((_CHUNK,), jnp.int32),
            pltpu.VMEM((_CHUNK,), jnp.int32),
            pltpu.VMEM((_TAIL,), jnp.int32),
            pltpu.VMEM((_CHUNK,), jnp.int32),
            pltpu.VMEM((_CHUNK,), jnp.int32),
            pltpu.VMEM((_CHUNK,), jnp.int32),
            pltpu.VMEM((_CHUNK,), jnp.int32),
            pltpu.VMEM((_CHUNK,), jnp.int32),
            pltpu.VMEM((_TAIL,), jnp.int32),
            pltpu.VMEM((_CHUNK, _F), jnp.float32),
            pltpu.VMEM((_CHUNK, _F), jnp.float32),
            pltpu.VMEM((_CHUNK, _F), jnp.float32),
            pltpu.VMEM((_CHUNK, _F), jnp.float32),
            pltpu.VMEM((_CHUNK, _F), jnp.float32),
            pltpu.VMEM((_TAIL, _F), jnp.float32),
            pltpu.VMEM_SHARED((_NPAD, _F), jnp.float32),
            pltpu.SemaphoreType.DMA,
            pltpu.SemaphoreType.DMA,
            pltpu.SemaphoreType.DMA,
        ],
    )


def _sc_agg(feat, src, dst):
    return _make_sc_agg()(feat, src, dst)

_R = 2000
_NB = _N // _R


_ROW = lambda i: (i, 0)
_FIX = lambda i: (0, 0)
_STAT = jax.ShapeDtypeStruct((1, _H), jnp.float32)


def _gin1_body(eps_s, x, p0, p1, w1, b1, w2, b2, g, be,
               h_out, slab, s_scr, q_scr):
    i = pl.program_id(0)

    @pl.when(i < _NB)
    def _compute():
        hin = (1.0 + eps_s[0]) * x[:] + p0[:] + p1[:]
        t = jnp.maximum(
            jnp.dot(hin, w1[:], preferred_element_type=jnp.float32)
            + b1[:], 0.0)
        u = jnp.maximum(
            jnp.dot(t, w2[:], preferred_element_type=jnp.float32)
            + b2[:], 0.0)
        slab[pl.ds(i * _R, _R), :] = u

        @pl.when(i == 0)
        def _init():
            s_scr[:] = jnp.zeros_like(s_scr)
            q_scr[:] = jnp.zeros_like(q_scr)

        s_scr[:] += jnp.sum(u, axis=0, keepdims=True)
        q_scr[:] += jnp.sum(u * u, axis=0, keepdims=True)

    @pl.when(i >= _NB)
    def _affine():
        m = s_scr[:] * (1.0 / _N)
        v = q_scr[:] * (1.0 / _N) - m * m
        a = g[:] * lax.rsqrt(v + 1e-5)
        j = i - _NB
        h_out[:] = slab[pl.ds(j * _R, _R), :] * a + (be[:] - m * a)


def _gin1(eps, xx, p0, p1, w1, b1, w2, b2, g, be):
    pin = lambda i: (jnp.minimum(i, _NB - 1), 0)
    out_map = lambda i: (jnp.maximum(i - _NB, 0), 0)
    return pl.pallas_call(
        _gin1_body,
        grid=(2 * _NB,),
        in_specs=[
            pl.BlockSpec(memory_space=pltpu.SMEM),
            pl.BlockSpec((_R, _F), pin),
            pl.BlockSpec((_R, _F), pin),
            pl.BlockSpec((_R, _F), pin),
            pl.BlockSpec((_F, _H), _FIX),
            pl.BlockSpec((1, _H), _FIX),
            pl.BlockSpec((_H, _H), _FIX),
            pl.BlockSpec((1, _H), _FIX),
            pl.BlockSpec((1, _H), _FIX),
            pl.BlockSpec((1, _H), _FIX),
        ],
        out_specs=pl.BlockSpec((_R, _H), out_map),
        out_shape=jax.ShapeDtypeStruct((_N, _H), jnp.float32),
        scratch_shapes=[pltpu.VMEM((_N, _H), jnp.float32),
                        pltpu.VMEM((1, _H), jnp.float32),
                        pltpu.VMEM((1, _H), jnp.float32)],
    )(eps, xx, p0, p1, w1, b1, w2, b2, g, be)


def _gin2_head_body(eps_s, h, p0, p1, w1, b1, w2, b2, g, be,
                    l1w, l1b, l2w, l2b, o, s_scr, q_scr):
    i = pl.program_id(0)
    hin = (1.0 + eps_s[0]) * h[:] + p0[:] + p1[:]
    t = jnp.maximum(
        jnp.dot(hin, w1[:], preferred_element_type=jnp.float32) + b1[:], 0.0)
    u = jnp.maximum(
        jnp.dot(t, w2[:], preferred_element_type=jnp.float32) + b2[:], 0.0)

    @pl.when(i == 0)
    def _init():
        s_scr[:] = jnp.zeros_like(s_scr)
        q_scr[:] = jnp.zeros_like(q_scr)

    s_scr[:] += jnp.sum(u, axis=0, keepdims=True)
    q_scr[:] += jnp.sum(u * u, axis=0, keepdims=True)

    @pl.when(i == _NB - 1)
    def _fin():
        m = s_scr[:] * (1.0 / _N)
        v = q_scr[:] * (1.0 / _N) - m * m
        a = g[:] * lax.rsqrt(v + 1e-5)
        pooled = m * a + (be[:] - m * a)
        tt = jnp.maximum(
            jnp.dot(pooled, l1w[:], preferred_element_type=jnp.float32)
            + l1b[:], 0.0)
        z = jnp.dot(tt, l2w[:], preferred_element_type=jnp.float32) + l2b[:]
        zm = jnp.max(z, axis=-1, keepdims=True)
        e = z - zm
        o[:] = e - jnp.log(jnp.sum(jnp.exp(e), axis=-1, keepdims=True))


def _gin2_head(eps, h, p0, p1, w1, b1, w2, b2, g, be, l1w, l1b, l2w, l2b):
    return pl.pallas_call(
        _gin2_head_body,
        grid=(_NB,),
        in_specs=[
            pl.BlockSpec(memory_space=pltpu.SMEM),
            pl.BlockSpec((_R, _F), _ROW),
            pl.BlockSpec((_R, _F), _ROW),
            pl.BlockSpec((_R, _F), _ROW),
            pl.BlockSpec((_F, _H), _FIX),
            pl.BlockSpec((1, _H), _FIX),
            pl.BlockSpec((_H, _H), _FIX),
            pl.BlockSpec((1, _H), _FIX),
            pl.BlockSpec((1, _H), _FIX),
            pl.BlockSpec((1, _H), _FIX),
            pl.BlockSpec((_H, _H), _FIX),
            pl.BlockSpec((1, _H), _FIX),
            pl.BlockSpec((_H, _C), _FIX),
            pl.BlockSpec((1, _C), _FIX),
        ],
        out_specs=pl.BlockSpec((1, _C), _FIX),
        out_shape=jax.ShapeDtypeStruct((1, _C), jnp.float32),
        scratch_shapes=[pltpu.VMEM((1, _H), jnp.float32),
                        pltpu.VMEM((1, _H), jnp.float32)],
    )(eps, h, p0, p1, w1, b1, w2, b2, g, be, l1w, l1b, l2w, l2b)


def kernel(x, edge_index, batch, c1_w1, c1_b1, c1_w2, c1_b2, c1_g, c1_be,
           c1_eps, c2_w1, c2_b1, c2_w2, c2_b2, c2_g, c2_be, c2_eps,
           lin1_w, lin1_b, lin2_w, lin2_b):
    src = edge_index[0]
    dst = edge_index[1]

    p1 = _sc_agg(x, src, dst)
    h1 = _gin1(c1_eps.reshape(1), x, p1[0], p1[1],
               c1_w1, c1_b1.reshape(1, -1),
               c1_w2, c1_b2.reshape(1, -1),
               c1_g.reshape(1, -1), c1_be.reshape(1, -1))

    p2 = _sc_agg(h1, src, dst)
    return _gin2_head(c2_eps.reshape(1), h1, p2[0], p2[1],
                      c2_w1, c2_b1.reshape(1, -1),
                      c2_w2, c2_b2.reshape(1, -1),
                      c2_g.reshape(1, -1), c2_be.reshape(1, -1),
                      lin1_w, lin1_b.reshape(1, -1),
                      lin2_w, lin2_b.reshape(1, -1))


# speedup vs baseline: 1.0088x; 1.0088x over previous
import functools

import jax
import jax.numpy as jnp
from jax import lax
from jax.experimental import pallas as pl
from jax.experimental.pallas import tpu as pltpu
from jax.experimental.pallas import tpu_sc as plsc

_N, _F, _H, _C = 10000, 128, 128, 10
_E = 320000
_NC, _NS = 2, 16
_NW = _NC * _NS
_EPT = _E // _NW
_CHUNK = 80
_NCH = _EPT // _CHUNK
_NPAD = 10112
_RPT = _NPAD // _NS
_NRING = 4
_NQ = _NCH // _NRING


def _agg_body(feat, src, dst, out,
              si0, si1, si2, si3, di0, di1, di2, di3,
              rows0, rows1, rows2, rows3,
              acc, gsem, ssem, isem):
    c = lax.axis_index("c")
    s = lax.axis_index("s")
    base = (c * _NS + s) * _EPT
    sis = (si0, si1, si2, si3)
    dis = (di0, di1, di2, di3)
    rows = (rows0, rows1, rows2, rows3)

    def sslice(k):
        return src.at[pl.ds(base + k * _CHUNK, _CHUNK)]

    def dslice(k):
        return dst.at[pl.ds(base + k * _CHUNK, _CHUNK)]

    for b in range(_NRING):
        pltpu.async_copy(sslice(b), sis[b], isem)
    for b in range(_NRING):
        pltpu.async_copy(dslice(b), dis[b], isem)
    for b in range(_NRING - 1):
        pltpu.make_async_copy(sslice(b), sis[b], isem).wait()
        pltpu.async_copy(feat.at[sis[b]], rows[b], gsem)

    zv = jnp.zeros((16,), jnp.float32)
    zbuf = rows[_NRING - 1]

    def zrow(r, carry):
        for k in range(8):
            zbuf[r, pl.ds(k * 16, 16)] = zv
        return carry

    lax.fori_loop(0, _CHUNK, zrow, 0)
    for t in range(_RPT // _CHUNK):
        pltpu.async_copy(zbuf, acc.at[pl.ds(s * _RPT + t * _CHUNK, _CHUNK)],
                         ssem)
    _nz = _RPT - (_RPT // _CHUNK) * _CHUNK
    pltpu.async_copy(zbuf.at[pl.ds(0, _nz)],
                     acc.at[pl.ds(s * _RPT + _RPT - _nz, _nz)], ssem)
    for t in range(_RPT // _CHUNK):
        pltpu.make_async_copy(
            zbuf, acc.at[pl.ds(s * _RPT + t * _CHUNK, _CHUNK)], ssem).wait()
    pltpu.make_async_copy(
        zbuf.at[pl.ds(0, _nz)],
        acc.at[pl.ds(s * _RPT + _RPT - _nz, _nz)], ssem).wait()
    plsc.subcore_barrier()

    pltpu.make_async_copy(sslice(_NRING - 1), sis[_NRING - 1], isem).wait()
    pltpu.async_copy(feat.at[sis[_NRING - 1]], rows[_NRING - 1], gsem)

    def round_(i, carry):
        j0 = _NRING * i
        for b in range(_NRING):
            k = j0 + b
            pltpu.make_async_copy(dslice(k), dis[b], isem).wait()
            pltpu.make_async_copy(feat.at[sis[b]], rows[b], gsem).wait()
            pltpu.async_copy(rows[b], acc.at[dis[b]], ssem, add=True)

            @pl.when(k + _NRING < _NCH)
            def _load_src(b=b, k=k):
                pltpu.async_copy(sslice(k + _NRING), sis[b], isem)

        for b in range(_NRING):
            k = j0 + b
            pltpu.make_async_copy(rows[b], acc.at[dis[b]], ssem).wait()

            @pl.when(k + _NRING < _NCH)
            def _issue(b=b, k=k):
                pltpu.async_copy(dslice(k + _NRING), dis[b], isem)
                pltpu.make_async_copy(sslice(k + _NRING), sis[b],
                                      isem).wait()
                pltpu.async_copy(feat.at[sis[b]], rows[b], gsem)

        return carry

    lax.fori_loop(0, _NQ, round_, 0)

    kl = _NQ * _NRING
    bl = kl % _NRING
    pltpu.make_async_copy(dslice(kl), dis[bl], isem).wait()
    pltpu.make_async_copy(feat.at[sis[bl]], rows[bl], gsem).wait()
    pltpu.async_copy(rows[bl], acc.at[dis[bl]], ssem, add=True)
    pltpu.make_async_copy(rows[bl], acc.at[dis[bl]], ssem).wait()

    plsc.subcore_barrier()
    pltpu.sync_copy(acc.at[pl.ds(s * _RPT, _RPT)],
                    out.at[c, pl.ds(s * _RPT, _RPT)])


@functools.cache
def _make_sc_agg():
    return pl.kernel(
        _agg_body,
        out_type=jax.ShapeDtypeStruct((_NC, _NPAD, _F), jnp.float32),
        mesh=plsc.VectorSubcoreMesh(core_axis_name="c", subcore_axis_name="s",
                                    num_cores=_NC, num_subcores=_NS),
        scratch_types=[
            pltpu.VMEM((_CHUNK,), jnp.int32),
            pltpu.VMEM((_CHUNK,), jnp.int32),
            pltpu.VMEM((_CHUNK,), jnp.int32),
            pltpu.VMEM((_CHUNK,), jnp.int32),
            pltpu.VMEM((_CHUNK,), jnp.int32),
            pltpu.VMEM((_CHUNK,), jnp.int32),
            pltpu.VMEM((_CHUNK,), jnp.int32),
            pltpu.VMEM((_CHUNK,), jnp.int32),
            pltpu.VMEM((_CHUNK, _F), jnp.float32),
            pltpu.VMEM((_CHUNK, _F), jnp.float32),
            pltpu.VMEM((_CHUNK, _F), jnp.float32),
            pltpu.VMEM((_CHUNK, _F), jnp.float32),
            pltpu.VMEM_SHARED((_NPAD, _F), jnp.float32),
            pltpu.SemaphoreType.DMA,
            pltpu.SemaphoreType.DMA,
            pltpu.SemaphoreType.DMA,
        ],
    )


def _sc_agg(feat, src, dst):
    return _make_sc_agg()(feat, src, dst)

_R = 2000
_NB = _N // _R


_ROW = lambda i: (i, 0)
_FIX = lambda i: (0, 0)
_STAT = jax.ShapeDtypeStruct((1, _H), jnp.float32)


def _gin1_body(eps_s, x, p0, p1, w1, b1, w2, b2, g, be,
               h_out, slab, s_scr, q_scr):
    i = pl.program_id(0)

    @pl.when(i < _NB)
    def _compute():
        hin = (1.0 + eps_s[0]) * x[:] + p0[:] + p1[:]
        t = jnp.maximum(
            jnp.dot(hin, w1[:], preferred_element_type=jnp.float32)
            + b1[:], 0.0)
        u = jnp.maximum(
            jnp.dot(t, w2[:], preferred_element_type=jnp.float32)
            + b2[:], 0.0)
        slab[pl.ds(i * _R, _R), :] = u

        @pl.when(i == 0)
        def _init():
            s_scr[:] = jnp.zeros_like(s_scr)
            q_scr[:] = jnp.zeros_like(q_scr)

        s_scr[:] += jnp.sum(u, axis=0, keepdims=True)
        q_scr[:] += jnp.sum(u * u, axis=0, keepdims=True)

    @pl.when(i >= _NB)
    def _affine():
        m = s_scr[:] * (1.0 / _N)
        v = q_scr[:] * (1.0 / _N) - m * m
        a = g[:] * lax.rsqrt(v + 1e-5)
        j = i - _NB
        h_out[:] = slab[pl.ds(j * _R, _R), :] * a + (be[:] - m * a)


def _gin1(eps, xx, p0, p1, w1, b1, w2, b2, g, be):
    pin = lambda i: (jnp.minimum(i, _NB - 1), 0)
    out_map = lambda i: (jnp.maximum(i - _NB, 0), 0)
    return pl.pallas_call(
        _gin1_body,
        grid=(2 * _NB,),
        in_specs=[
            pl.BlockSpec(memory_space=pltpu.SMEM),
            pl.BlockSpec((_R, _F), pin),
            pl.BlockSpec((_R, _F), pin),
            pl.BlockSpec((_R, _F), pin),
            pl.BlockSpec((_F, _H), _FIX),
            pl.BlockSpec((1, _H), _FIX),
            pl.BlockSpec((_H, _H), _FIX),
            pl.BlockSpec((1, _H), _FIX),
            pl.BlockSpec((1, _H), _FIX),
            pl.BlockSpec((1, _H), _FIX),
        ],
        out_specs=pl.BlockSpec((_R, _H), out_map),
        out_shape=jax.ShapeDtypeStruct((_N, _H), jnp.float32),
        scratch_shapes=[pltpu.VMEM((_N, _H), jnp.float32),
                        pltpu.VMEM((1, _H), jnp.float32),
                        pltpu.VMEM((1, _H), jnp.float32)],
    )(eps, xx, p0, p1, w1, b1, w2, b2, g, be)


def _gin2_head_body(eps_s, h, p0, p1, w1, b1, w2, b2, g, be,
                    l1w, l1b, l2w, l2b, o, s_scr, q_scr):
    i = pl.program_id(0)
    hin = (1.0 + eps_s[0]) * h[:] + p0[:] + p1[:]
    t = jnp.maximum(
        jnp.dot(hin, w1[:], preferred_element_type=jnp.float32) + b1[:], 0.0)
    u = jnp.maximum(
        jnp.dot(t, w2[:], preferred_element_type=jnp.float32) + b2[:], 0.0)

    @pl.when(i == 0)
    def _init():
        s_scr[:] = jnp.zeros_like(s_scr)
        q_scr[:] = jnp.zeros_like(q_scr)

    s_scr[:] += jnp.sum(u, axis=0, keepdims=True)
    q_scr[:] += jnp.sum(u * u, axis=0, keepdims=True)

    @pl.when(i == _NB - 1)
    def _fin():
        m = s_scr[:] * (1.0 / _N)
        v = q_scr[:] * (1.0 / _N) - m * m
        a = g[:] * lax.rsqrt(v + 1e-5)
        pooled = m * a + (be[:] - m * a)
        tt = jnp.maximum(
            jnp.dot(pooled, l1w[:], preferred_element_type=jnp.float32)
            + l1b[:], 0.0)
        z = jnp.dot(tt, l2w[:], preferred_element_type=jnp.float32) + l2b[:]
        zm = jnp.max(z, axis=-1, keepdims=True)
        e = z - zm
        o[:] = e - jnp.log(jnp.sum(jnp.exp(e), axis=-1, keepdims=True))


def _gin2_head(eps, h, p0, p1, w1, b1, w2, b2, g, be, l1w, l1b, l2w, l2b):
    return pl.pallas_call(
        _gin2_head_body,
        grid=(_NB,),
        in_specs=[
            pl.BlockSpec(memory_space=pltpu.SMEM),
            pl.BlockSpec((_R, _F), _ROW),
            pl.BlockSpec((_R, _F), _ROW),
            pl.BlockSpec((_R, _F), _ROW),
            pl.BlockSpec((_F, _H), _FIX),
            pl.BlockSpec((1, _H), _FIX),
            pl.BlockSpec((_H, _H), _FIX),
            pl.BlockSpec((1, _H), _FIX),
            pl.BlockSpec((1, _H), _FIX),
            pl.BlockSpec((1, _H), _FIX),
            pl.BlockSpec((_H, _H), _FIX),
            pl.BlockSpec((1, _H), _FIX),
            pl.BlockSpec((_H, _C), _FIX),
            pl.BlockSpec((1, _C), _FIX),
        ],
        out_specs=pl.BlockSpec((1, _C), _FIX),
        out_shape=jax.ShapeDtypeStruct((1, _C), jnp.float32),
        scratch_shapes=[pltpu.VMEM((1, _H), jnp.float32),
                        pltpu.VMEM((1, _H), jnp.float32)],
    )(eps, h, p0, p1, w1, b1, w2, b2, g, be, l1w, l1b, l2w, l2b)


def kernel(x, edge_index, batch, c1_w1, c1_b1, c1_w2, c1_b2, c1_g, c1_be,
           c1_eps, c2_w1, c2_b1, c2_w2, c2_b2, c2_g, c2_be, c2_eps,
           lin1_w, lin1_b, lin2_w, lin2_b):
    src = edge_index[0]
    dst = edge_index[1]

    p1 = _sc_agg(x, src, dst)
    h1 = _gin1(c1_eps.reshape(1), x, p1[0], p1[1],
               c1_w1, c1_b1.reshape(1, -1),
               c1_w2, c1_b2.reshape(1, -1),
               c1_g.reshape(1, -1), c1_be.reshape(1, -1))

    p2 = _sc_agg(h1, src, dst)
    return _gin2_head(c2_eps.reshape(1), h1, p2[0], p2[1],
                      c2_w1, c2_b1.reshape(1, -1),
                      c2_w2, c2_b2.reshape(1, -1),
                      c2_g.reshape(1, -1), c2_be.reshape(1, -1),
                      lin1_w, lin1_b.reshape(1, -1),
                      lin2_w, lin2_b.reshape(1, -1))


# speedup vs baseline: 1.0146x; 1.0058x over previous
import functools

import jax
import jax.numpy as jnp
from jax import lax
from jax.experimental import pallas as pl
from jax.experimental.pallas import tpu as pltpu
from jax.experimental.pallas import tpu_sc as plsc

_N, _F, _H, _C = 10000, 128, 128, 10
_E = 320000
_NC, _NS = 2, 16
_NW = _NC * _NS
_EPT = _E // _NW
_CHUNK = 80
_NCH = _EPT // _CHUNK
_NPAD = 10112
_RPT = _NPAD // _NS
_NRING = 4
_NQ = _NCH // _NRING


def _agg_body(feat, src, dst, out,
              si0, si1, si2, si3, di0, di1, di2, di3,
              rows0, rows1, rows2, rows3,
              acc, gsem, ssem, isem):
    c = lax.axis_index("c")
    s = lax.axis_index("s")
    base = (c * _NS + s) * _EPT
    sis = (si0, si1, si2, si3)
    dis = (di0, di1, di2, di3)
    rows = (rows0, rows1, rows2, rows3)

    def sslice(k):
        return src.at[pl.ds(base + k * _CHUNK, _CHUNK)]

    def dslice(k):
        return dst.at[pl.ds(base + k * _CHUNK, _CHUNK)]

    for b in range(_NRING):
        pltpu.async_copy(sslice(b), sis[b], isem)
    for b in range(_NRING):
        pltpu.async_copy(dslice(b), dis[b], isem)
    for b in range(_NRING - 1):
        pltpu.make_async_copy(sslice(b), sis[b], isem).wait()
        pltpu.async_copy(feat.at[sis[b]], rows[b], gsem)

    zv = jnp.zeros((16,), jnp.float32)
    zbuf = rows[_NRING - 1]

    def zrow(r, carry):
        for k in range(8):
            zbuf[r, pl.ds(k * 16, 16)] = zv
        return carry

    lax.fori_loop(0, _CHUNK, zrow, 0)
    for t in range(_RPT // _CHUNK):
        pltpu.async_copy(zbuf, acc.at[pl.ds(s * _RPT + t * _CHUNK, _CHUNK)],
                         ssem)
    _nz = _RPT - (_RPT // _CHUNK) * _CHUNK
    pltpu.async_copy(zbuf.at[pl.ds(0, _nz)],
                     acc.at[pl.ds(s * _RPT + _RPT - _nz, _nz)], ssem)
    for t in range(_RPT // _CHUNK):
        pltpu.make_async_copy(
            zbuf, acc.at[pl.ds(s * _RPT + t * _CHUNK, _CHUNK)], ssem).wait()
    pltpu.make_async_copy(
        zbuf.at[pl.ds(0, _nz)],
        acc.at[pl.ds(s * _RPT + _RPT - _nz, _nz)], ssem).wait()
    plsc.subcore_barrier()

    pltpu.make_async_copy(sslice(_NRING - 1), sis[_NRING - 1], isem).wait()
    pltpu.async_copy(feat.at[sis[_NRING - 1]], rows[_NRING - 1], gsem)

    def round_(i, carry):
        j0 = _NRING * i
        for b in range(_NRING):
            k = j0 + b
            pltpu.make_async_copy(dslice(k), dis[b], isem).wait()
            pltpu.make_async_copy(feat.at[sis[b]], rows[b], gsem).wait()
            pltpu.async_copy(rows[b], acc.at[dis[b]], ssem, add=True)

            @pl.when(k + _NRING < _NCH)
            def _load_src(b=b, k=k):
                pltpu.async_copy(sslice(k + _NRING), sis[b], isem)

        for b in range(_NRING):
            k = j0 + b
            pltpu.make_async_copy(rows[b], acc.at[dis[b]], ssem).wait()

            @pl.when(k + _NRING < _NCH)
            def _issue(b=b, k=k):
                pltpu.async_copy(dslice(k + _NRING), dis[b], isem)
                pltpu.make_async_copy(sslice(k + _NRING), sis[b],
                                      isem).wait()
                pltpu.async_copy(feat.at[sis[b]], rows[b], gsem)

        return carry

    lax.fori_loop(0, _NQ, round_, 0)

    kl = _NQ * _NRING
    bl = kl % _NRING
    pltpu.make_async_copy(dslice(kl), dis[bl], isem).wait()
    pltpu.make_async_copy(feat.at[sis[bl]], rows[bl], gsem).wait()
    pltpu.async_copy(rows[bl], acc.at[dis[bl]], ssem, add=True)
    pltpu.make_async_copy(rows[bl], acc.at[dis[bl]], ssem).wait()

    plsc.subcore_barrier()
    pltpu.sync_copy(acc.at[pl.ds(s * _RPT, _RPT)],
                    out.at[c, pl.ds(s * _RPT, _RPT)])


@functools.cache
def _make_sc_agg():
    return pl.kernel(
        _agg_body,
        out_type=jax.ShapeDtypeStruct((_NC, _NPAD, _F), jnp.float32),
        mesh=plsc.VectorSubcoreMesh(core_axis_name="c", subcore_axis_name="s",
                                    num_cores=_NC, num_subcores=_NS),
        scratch_types=[
            pltpu.VMEM((_CHUNK,), jnp.int32),
            pltpu.VMEM((_CHUNK,), jnp.int32),
            pltpu.VMEM((_CHUNK,), jnp.int32),
            pltpu.VMEM((_CHUNK,), jnp.int32),
            pltpu.VMEM((_CHUNK,), jnp.int32),
            pltpu.VMEM((_CHUNK,), jnp.int32),
            pltpu.VMEM((_CHUNK,), jnp.int32),
            pltpu.VMEM((_CHUNK,), jnp.int32),
            pltpu.VMEM((_CHUNK, _F), jnp.float32),
            pltpu.VMEM((_CHUNK, _F), jnp.float32),
            pltpu.VMEM((_CHUNK, _F), jnp.float32),
            pltpu.VMEM((_CHUNK, _F), jnp.float32),
            pltpu.VMEM_SHARED((_NPAD, _F), jnp.float32),
            pltpu.SemaphoreType.DMA,
            pltpu.SemaphoreType.DMA,
            pltpu.SemaphoreType.DMA,
        ],
    )


def _sc_agg(feat, src, dst):
    return _make_sc_agg()(feat, src, dst)

_R = 2000
_NB = _N // _R


_ROW = lambda i: (i, 0)
_FIX = lambda i: (0, 0)


def _gin1_body(eps_s, x, p0, p1, w1, b1, w2, b2, g, be,
               h_out, slab, s_scr, q_scr):
    i = pl.program_id(0)

    @pl.when(i < _NB)
    def _compute():
        hin = (1.0 + eps_s[0]) * x[:] + p0[:] + p1[:]
        t = jnp.maximum(
            jnp.dot(hin, w1[:], preferred_element_type=jnp.float32)
            + b1[:], 0.0)
        u = jnp.maximum(
            jnp.dot(t, w2[:], preferred_element_type=jnp.float32)
            + b2[:], 0.0)
        slab[pl.ds(i * _R, _R), :] = u

        @pl.when(i == 0)
        def _init():
            s_scr[:] = jnp.zeros_like(s_scr)
            q_scr[:] = jnp.zeros_like(q_scr)

        s_scr[:] += jnp.sum(u, axis=0, keepdims=True)
        q_scr[:] += jnp.sum(u * u, axis=0, keepdims=True)

    @pl.when(i >= _NB)
    def _affine():
        m = s_scr[:] * (1.0 / _N)
        v = q_scr[:] * (1.0 / _N) - m * m
        a = g[:] * lax.rsqrt(v + 1e-5)
        j = i - _NB
        h_out[:] = slab[pl.ds(j * _R, _R), :] * a + (be[:] - m * a)


def _gin1(eps, xx, p0, p1, w1, b1, w2, b2, g, be):
    pin = lambda i: (jnp.minimum(i, _NB - 1), 0)
    out_map = lambda i: (jnp.maximum(i - _NB, 0), 0)
    return pl.pallas_call(
        _gin1_body,
        grid=(2 * _NB,),
        in_specs=[
            pl.BlockSpec(memory_space=pltpu.SMEM),
            pl.BlockSpec((_R, _F), pin),
            pl.BlockSpec((_R, _F), pin),
            pl.BlockSpec((_R, _F), pin),
            pl.BlockSpec((_F, _H), _FIX),
            pl.BlockSpec((1, _H), _FIX),
            pl.BlockSpec((_H, _H), _FIX),
            pl.BlockSpec((1, _H), _FIX),
            pl.BlockSpec((1, _H), _FIX),
            pl.BlockSpec((1, _H), _FIX),
        ],
        out_specs=pl.BlockSpec((_R, _H), out_map),
        out_shape=jax.ShapeDtypeStruct((_N, _H), jnp.float32),
        scratch_shapes=[pltpu.VMEM((_N, _H), jnp.float32),
                        pltpu.VMEM((1, _H), jnp.float32),
                        pltpu.VMEM((1, _H), jnp.float32)],
    )(eps, xx, p0, p1, w1, b1, w2, b2, g, be)


def _gin2_head_body(eps_s, h, p0, p1, w1, b1, w2, b2, g, be,
                    l1w, l1b, l2w, l2b, o, s_scr, q_scr):
    i = pl.program_id(0)
    hin = (1.0 + eps_s[0]) * h[:] + p0[:] + p1[:]
    t = jnp.maximum(
        jnp.dot(hin, w1[:], preferred_element_type=jnp.float32) + b1[:], 0.0)
    u = jnp.maximum(
        jnp.dot(t, w2[:], preferred_element_type=jnp.float32) + b2[:], 0.0)

    @pl.when(i == 0)
    def _init():
        s_scr[:] = jnp.zeros_like(s_scr)
        q_scr[:] = jnp.zeros_like(q_scr)

    s_scr[:] += jnp.sum(u, axis=0, keepdims=True)
    q_scr[:] += jnp.sum(u * u, axis=0, keepdims=True)

    @pl.when(i == _NB - 1)
    def _fin():
        m = s_scr[:] * (1.0 / _N)
        v = q_scr[:] * (1.0 / _N) - m * m
        a = g[:] * lax.rsqrt(v + 1e-5)
        pooled = m * a + (be[:] - m * a)
        tt = jnp.maximum(
            jnp.dot(pooled, l1w[:], preferred_element_type=jnp.float32)
            + l1b[:], 0.0)
        z = jnp.dot(tt, l2w[:], preferred_element_type=jnp.float32) + l2b[:]
        zm = jnp.max(z, axis=-1, keepdims=True)
        e = z - zm
        o[:] = e - jnp.log(jnp.sum(jnp.exp(e), axis=-1, keepdims=True))


def _gin2_head(eps, h, p0, p1, w1, b1, w2, b2, g, be, l1w, l1b, l2w, l2b):
    return pl.pallas_call(
        _gin2_head_body,
        grid=(_NB,),
        in_specs=[
            pl.BlockSpec(memory_space=pltpu.SMEM),
            pl.BlockSpec((_R, _F), _ROW),
            pl.BlockSpec((_R, _F), _ROW),
            pl.BlockSpec((_R, _F), _ROW),
            pl.BlockSpec((_F, _H), _FIX),
            pl.BlockSpec((1, _H), _FIX),
            pl.BlockSpec((_H, _H), _FIX),
            pl.BlockSpec((1, _H), _FIX),
            pl.BlockSpec((1, _H), _FIX),
            pl.BlockSpec((1, _H), _FIX),
            pl.BlockSpec((_H, _H), _FIX),
            pl.BlockSpec((1, _H), _FIX),
            pl.BlockSpec((_H, _C), _FIX),
            pl.BlockSpec((1, _C), _FIX),
        ],
        out_specs=pl.BlockSpec((1, _C), _FIX),
        out_shape=jax.ShapeDtypeStruct((1, _C), jnp.float32),
        scratch_shapes=[pltpu.VMEM((1, _H), jnp.float32),
                        pltpu.VMEM((1, _H), jnp.float32)],
    )(eps, h, p0, p1, w1, b1, w2, b2, g, be, l1w, l1b, l2w, l2b)


def kernel(x, edge_index, batch, c1_w1, c1_b1, c1_w2, c1_b2, c1_g, c1_be,
           c1_eps, c2_w1, c2_b1, c2_w2, c2_b2, c2_g, c2_be, c2_eps,
           lin1_w, lin1_b, lin2_w, lin2_b):
    src = edge_index[0]
    dst = edge_index[1]

    p1 = _sc_agg(x, src, dst)
    h1 = _gin1(c1_eps.reshape(1), x, p1[0], p1[1],
               c1_w1, c1_b1.reshape(1, -1),
               c1_w2, c1_b2.reshape(1, -1),
               c1_g.reshape(1, -1), c1_be.reshape(1, -1))

    p2 = _sc_agg(h1, src, dst)
    return _gin2_head(c2_eps.reshape(1), h1, p2[0], p2[1],
                      c2_w1, c2_b1.reshape(1, -1),
                      c2_w2, c2_b2.reshape(1, -1),
                      c2_g.reshape(1, -1), c2_be.reshape(1, -1),
                      lin1_w, lin1_b.reshape(1, -1),
                      lin2_w, lin2_b.reshape(1, -1))
